# baseline (speedup 1.0000x reference)
import jax
import jax.numpy as jnp
from jax.experimental import pallas as pl
from jax.experimental.pallas import tpu as pltpu

EPS = 1e-6
NSTREAM = 1
SUB = 4096
CHUNK = NSTREAM * SUB
SROWS = SUB // 128


def _kernel(act_ref, cand_ref, inp_ref, *rest):
    enc_refs = rest[:NSTREAM]
    ins_ref, mask_ref, out_ref, acc_ref = rest[NSTREAM:]
    b = pl.program_id(0)
    i = pl.program_id(1)
    nc = pl.num_programs(1)

    @pl.when(i == 0)
    def _init():
        acc_ref[...] = jnp.zeros_like(acc_ref)

    a = act_ref[b]
    si = (a // 128) % 8
    lj = a % 128
    win = inp_ref[0, :, 0]
    c = win.shape[0]
    sub_ids = jax.lax.broadcasted_iota(jnp.int32, win.shape, 1)
    lane_ids = jax.lax.broadcasted_iota(jnp.int32, win.shape, 2)
    hit = (sub_ids == si) & (lane_ids == lj)
    sel = jnp.sum(jnp.where(hit, win, 0.0), axis=(1, 2))

    scale = 1.0 / jnp.sqrt(jnp.float32(c))
    l_parts = []
    for s in range(NSTREAM):
        e = enc_refs[s][0]
        lg = jax.lax.dot_general(
            sel[None, :], e, (((1,), (1,)), ((), ())),
            preferred_element_type=jnp.float32)
        l_parts.append(lg.reshape(SROWS, 128))
    l2 = jnp.concatenate(l_parts, axis=0) * scale
    pred = jax.nn.sigmoid(l2)

    m = (mask_ref[0] > 0.5).astype(jnp.float32)
    g = (ins_ref[0] > 0.5).astype(jnp.float32)

    p = pred * m
    t = g * m
    pt = p * t + (1.0 - p) * (1.0 - t)
    one_m_pt = 1.0 - pt
    focal = -(one_m_pt * one_m_pt) * jnp.log(pt + EPS) * m

    acc_ref[0, :] += jnp.sum(focal, axis=0)
    acc_ref[1, :] += jnp.sum(p * t, axis=0)
    acc_ref[2, :] += jnp.sum(p, axis=0)
    acc_ref[3, :] += jnp.sum(t, axis=0)
    acc_ref[4, :] += jnp.sum(m, axis=0)

    @pl.when(i == nc - 1)
    def _fin():
        focal_sum = jnp.sum(acc_ref[0, :])
        inter = jnp.sum(acc_ref[1, :])
        sum_p = jnp.sum(acc_ref[2, :])
        sum_t = jnp.sum(acc_ref[3, :])
        mask_sum = jnp.sum(acc_ref[4, :])
        focal_loss = focal_sum / (mask_sum + EPS)
        dice_loss = 1.0 - (2.0 * inter + EPS) / (sum_p + sum_t + EPS)
        loss_atten = (0.5 * focal_loss + dice_loss) * sum_t
        out_ref[0, 0, :] = jnp.full((128,), loss_atten / (mask_sum + EPS))


def kernel(input, encode, ins_seg, mask, actions, candidate_idx):
    b, c, h, w = input.shape
    hw = h * w
    n_ins = ins_seg.shape[1]
    nc = hw // CHUNK

    inp4 = input.reshape(b, c, h // 8, 8, w)
    ins_rows = ins_seg.reshape(b * n_ins, hw // 128, 128)
    mask_rows = mask.reshape(b, hw // 128, 128)

    def enc_spec(s):
        return pl.BlockSpec(
            (1, SUB, c),
            lambda bi, ci, act, cand, s=s: (bi, ci * NSTREAM + s, 0))

    grid_spec = pltpu.PrefetchScalarGridSpec(
        num_scalar_prefetch=2,
        grid=(b, nc),
        in_specs=[
            pl.BlockSpec((1, c, 1, 8, 128),
                         lambda bi, ci, act, cand:
                         (bi, 0, act[bi] // 1024, 0, 0)),
            *[enc_spec(s) for s in range(NSTREAM)],
            pl.BlockSpec((1, CHUNK // 128, 128),
                         lambda bi, ci, act, cand:
                         (bi * n_ins + cand[bi], ci, 0)),
            pl.BlockSpec((1, CHUNK // 128, 128),
                         lambda bi, ci, act, cand: (bi, ci, 0)),
        ],
        out_specs=pl.BlockSpec((1, 1, 128),
                               lambda bi, ci, act, cand: (bi, 0, 0)),
        scratch_shapes=[pltpu.VMEM((8, 128), jnp.float32)],
    )

    out = pl.pallas_call(
        _kernel,
        grid_spec=grid_spec,
        out_shape=jax.ShapeDtypeStruct((b, 1, 128), jnp.float32),
    )(actions, candidate_idx, inp4,
      *([encode] * NSTREAM), ins_rows, mask_rows)
    return out[:, 0, 0]


# speedup vs baseline: 1.1339x; 1.1339x over previous
import jax
import jax.numpy as jnp
from jax.experimental import pallas as pl
from jax.experimental.pallas import tpu as pltpu

EPS = 1e-6
NSTREAM = 1
SUB = 16384
CHUNK = NSTREAM * SUB
SROWS = SUB // 128


def _kernel(act_ref, cand_ref, inp_ref, *rest):
    enc_refs = rest[:NSTREAM]
    ins_ref, mask_ref, out_ref, acc_ref = rest[NSTREAM:]
    b = pl.program_id(0)
    i = pl.program_id(1)
    nc = pl.num_programs(1)

    @pl.when(i == 0)
    def _init():
        acc_ref[...] = jnp.zeros_like(acc_ref)

    a = act_ref[b]
    si = (a // 128) % 8
    lj = a % 128
    win = inp_ref[0, :, 0]
    c = win.shape[0]
    sub_ids = jax.lax.broadcasted_iota(jnp.int32, win.shape, 1)
    lane_ids = jax.lax.broadcasted_iota(jnp.int32, win.shape, 2)
    hit = (sub_ids == si) & (lane_ids == lj)
    sel = jnp.sum(jnp.where(hit, win, 0.0), axis=(1, 2))

    scale = 1.0 / jnp.sqrt(jnp.float32(c))
    l_parts = []
    for s in range(NSTREAM):
        e = enc_refs[s][0]
        lg = jax.lax.dot_general(
            sel[None, :], e, (((1,), (1,)), ((), ())),
            preferred_element_type=jnp.float32)
        l_parts.append(lg.reshape(SROWS, 128))
    l2 = jnp.concatenate(l_parts, axis=0) * scale
    pred = jax.nn.sigmoid(l2)

    m = (mask_ref[0] > 0.5).astype(jnp.float32)
    g = (ins_ref[0] > 0.5).astype(jnp.float32)

    p = pred * m
    t = g * m
    pt = p * t + (1.0 - p) * (1.0 - t)
    one_m_pt = 1.0 - pt
    focal = -(one_m_pt * one_m_pt) * jnp.log(pt + EPS) * m

    acc_ref[0, :] += jnp.sum(focal, axis=0)
    acc_ref[1, :] += jnp.sum(p * t, axis=0)
    acc_ref[2, :] += jnp.sum(p, axis=0)
    acc_ref[3, :] += jnp.sum(t, axis=0)
    acc_ref[4, :] += jnp.sum(m, axis=0)

    @pl.when(i == nc - 1)
    def _fin():
        focal_sum = jnp.sum(acc_ref[0, :])
        inter = jnp.sum(acc_ref[1, :])
        sum_p = jnp.sum(acc_ref[2, :])
        sum_t = jnp.sum(acc_ref[3, :])
        mask_sum = jnp.sum(acc_ref[4, :])
        focal_loss = focal_sum / (mask_sum + EPS)
        dice_loss = 1.0 - (2.0 * inter + EPS) / (sum_p + sum_t + EPS)
        loss_atten = (0.5 * focal_loss + dice_loss) * sum_t
        out_ref[0, 0, :] = jnp.full((128,), loss_atten / (mask_sum + EPS))


def kernel(input, encode, ins_seg, mask, actions, candidate_idx):
    b, c, h, w = input.shape
    hw = h * w
    n_ins = ins_seg.shape[1]
    nc = hw // CHUNK

    inp4 = input.reshape(b, c, h // 8, 8, w)
    ins_rows = ins_seg.reshape(b * n_ins, hw // 128, 128)
    mask_rows = mask.reshape(b, hw // 128, 128)

    def enc_spec(s):
        return pl.BlockSpec(
            (1, SUB, c),
            lambda bi, ci, act, cand, s=s: (bi, ci * NSTREAM + s, 0))

    grid_spec = pltpu.PrefetchScalarGridSpec(
        num_scalar_prefetch=2,
        grid=(b, nc),
        in_specs=[
            pl.BlockSpec((1, c, 1, 8, 128),
                         lambda bi, ci, act, cand:
                         (bi, 0, act[bi] // 1024, 0, 0)),
            *[enc_spec(s) for s in range(NSTREAM)],
            pl.BlockSpec((1, CHUNK // 128, 128),
                         lambda bi, ci, act, cand:
                         (bi * n_ins + cand[bi], ci, 0)),
            pl.BlockSpec((1, CHUNK // 128, 128),
                         lambda bi, ci, act, cand: (bi, ci, 0)),
        ],
        out_specs=pl.BlockSpec((1, 1, 128),
                               lambda bi, ci, act, cand: (bi, 0, 0)),
        scratch_shapes=[pltpu.VMEM((8, 128), jnp.float32)],
    )

    out = pl.pallas_call(
        _kernel,
        grid_spec=grid_spec,
        out_shape=jax.ShapeDtypeStruct((b, 1, 128), jnp.float32),
    )(actions, candidate_idx, inp4,
      *([encode] * NSTREAM), ins_rows, mask_rows)
    return out[:, 0, 0]


# speedup vs baseline: 1.1739x; 1.0353x over previous
import jax
import jax.numpy as jnp
from jax.experimental import pallas as pl
from jax.experimental.pallas import tpu as pltpu

EPS = 1e-6
NSTREAM = 1
SUB = 8192
CHUNK = NSTREAM * SUB
SROWS = SUB // 128


def _kernel(act_ref, cand_ref, inp_ref, *rest):
    enc_refs = rest[:NSTREAM]
    ins_ref, mask_ref, out_ref, acc_ref = rest[NSTREAM:]
    b = pl.program_id(0)
    i = pl.program_id(1)
    nc = pl.num_programs(1)

    @pl.when(i == 0)
    def _init():
        acc_ref[...] = jnp.zeros_like(acc_ref)

    a = act_ref[b]
    si = (a // 128) % 8
    lj = a % 128
    win = inp_ref[0, :, 0]
    c = win.shape[0]
    sub_ids = jax.lax.broadcasted_iota(jnp.int32, win.shape, 1)
    lane_ids = jax.lax.broadcasted_iota(jnp.int32, win.shape, 2)
    hit = (sub_ids == si) & (lane_ids == lj)
    sel = jnp.sum(jnp.where(hit, win, 0.0), axis=(1, 2))

    scale = 1.0 / jnp.sqrt(jnp.float32(c))
    l_parts = []
    for s in range(NSTREAM):
        e = enc_refs[s][0]
        lg = jax.lax.dot_general(
            sel[None, :], e, (((1,), (1,)), ((), ())),
            preferred_element_type=jnp.float32)
        l_parts.append(lg.reshape(SROWS, 128))
    l2 = jnp.concatenate(l_parts, axis=0) * scale
    pred = jax.nn.sigmoid(l2)

    m = (mask_ref[0] > 0.5).astype(jnp.float32)
    g = (ins_ref[0] > 0.5).astype(jnp.float32)

    p = pred * m
    t = g * m
    pt = p * t + (1.0 - p) * (1.0 - t)
    one_m_pt = 1.0 - pt
    focal = -(one_m_pt * one_m_pt) * jnp.log(pt + EPS) * m

    acc_ref[0, :] += jnp.sum(focal, axis=0)
    acc_ref[1, :] += jnp.sum(p * t, axis=0)
    acc_ref[2, :] += jnp.sum(p, axis=0)
    acc_ref[3, :] += jnp.sum(t, axis=0)
    acc_ref[4, :] += jnp.sum(m, axis=0)

    @pl.when(i == nc - 1)
    def _fin():
        focal_sum = jnp.sum(acc_ref[0, :])
        inter = jnp.sum(acc_ref[1, :])
        sum_p = jnp.sum(acc_ref[2, :])
        sum_t = jnp.sum(acc_ref[3, :])
        mask_sum = jnp.sum(acc_ref[4, :])
        focal_loss = focal_sum / (mask_sum + EPS)
        dice_loss = 1.0 - (2.0 * inter + EPS) / (sum_p + sum_t + EPS)
        loss_atten = (0.5 * focal_loss + dice_loss) * sum_t
        out_ref[0, 0, :] = jnp.full((128,), loss_atten / (mask_sum + EPS))


def kernel(input, encode, ins_seg, mask, actions, candidate_idx):
    b, c, h, w = input.shape
    hw = h * w
    n_ins = ins_seg.shape[1]
    nc = hw // CHUNK

    inp4 = input.reshape(b, c, h // 8, 8, w)
    ins_rows = ins_seg.reshape(b * n_ins, hw // 128, 128)
    mask_rows = mask.reshape(b, hw // 128, 128)

    def enc_spec(s):
        return pl.BlockSpec(
            (1, SUB, c),
            lambda bi, ci, act, cand, s=s: (bi, ci * NSTREAM + s, 0))

    grid_spec = pltpu.PrefetchScalarGridSpec(
        num_scalar_prefetch=2,
        grid=(b, nc),
        in_specs=[
            pl.BlockSpec((1, c, 1, 8, 128),
                         lambda bi, ci, act, cand:
                         (bi, 0, act[bi] // 1024, 0, 0)),
            *[enc_spec(s) for s in range(NSTREAM)],
            pl.BlockSpec((1, CHUNK // 128, 128),
                         lambda bi, ci, act, cand:
                         (bi * n_ins + cand[bi], ci, 0)),
            pl.BlockSpec((1, CHUNK // 128, 128),
                         lambda bi, ci, act, cand: (bi, ci, 0)),
        ],
        out_specs=pl.BlockSpec((1, 1, 128),
                               lambda bi, ci, act, cand: (bi, 0, 0)),
        scratch_shapes=[pltpu.VMEM((8, 128), jnp.float32)],
    )

    out = pl.pallas_call(
        _kernel,
        grid_spec=grid_spec,
        out_shape=jax.ShapeDtypeStruct((b, 1, 128), jnp.float32),
    )(actions, candidate_idx, inp4,
      *([encode] * NSTREAM), ins_rows, mask_rows)
    return out[:, 0, 0]
